# baseline (speedup 1.0000x reference)
import functools

import jax
import jax.numpy as jnp
from jax import lax
from jax.experimental import pallas as pl
from jax.experimental.pallas import tpu as pltpu
from jax.experimental.pallas import tpu_sc as plsc

H = 768
E = 128
I = 1856
IS = 3712
TOPK = 6
RSF = 2.5
T = 2048

BLK = 128
NB = 224
PAD = NB * BLK

NC = 2
NS = 16
NW = NC * NS
A = T * TOPK
A_W = A // NW
CH = 64
NCH = A_W // CH


def _router_body(x_ref, rw_ref, bias_ref, w_ref, pos_ref, be_ref):
    x = x_ref[...]
    rw = rw_ref[...]
    logits = lax.dot_general(
        x, rw, (((1,), (1,)), ((), ())),
        preferred_element_type=jnp.float32)
    scores = jax.nn.sigmoid(logits)
    sfc = scores + bias_ref[...]

    lane = lax.broadcasted_iota(jnp.int32, (T, E), 1)
    cur = sfc
    onehots = []
    ws = []
    for _ in range(TOPK):
        m = jnp.max(cur, axis=1, keepdims=True)
        idx = jnp.min(jnp.where(cur == m, lane, E), axis=1, keepdims=True)
        oh = lane == idx
        onehots.append(oh)
        ws.append(jnp.sum(jnp.where(oh, scores, 0.0), axis=1, keepdims=True))
        cur = jnp.where(oh, -1e30, cur)

    w = jnp.concatenate(ws, axis=1)
    denom = jnp.sum(w, axis=1, keepdims=True) + 1e-20
    w_ref[...] = w / denom * RSF

    hist = jnp.zeros((T, E), jnp.float32)
    for oh in onehots:
        hist = hist + oh.astype(jnp.float32)

    r128 = lax.broadcasted_iota(jnp.int32, (BLK, BLK), 0)
    c128 = lax.broadcasted_iota(jnp.int32, (BLK, BLK), 1)
    ltri = (r128 > c128).astype(jnp.float32)
    carry = jnp.zeros((1, E), jnp.float32)
    chunks = []
    for c in range(T // BLK):
        h_c = hist[c * BLK:(c + 1) * BLK, :]
        e_c = lax.dot_general(ltri, h_c, (((1,), (0,)), ((), ())),
                              preferred_element_type=jnp.float32) + carry
        chunks.append(e_c)
        carry = carry + jnp.sum(h_c, axis=0, keepdims=True)
    excl = jnp.concatenate(chunks, axis=0)
    counts = carry

    nblk = jnp.ceil(counts / BLK)
    re_ = lax.broadcasted_iota(jnp.int32, (E, E), 0)
    ce_ = lax.broadcasted_iota(jnp.int32, (E, E), 1)
    utri = (re_ < ce_).astype(jnp.float32)
    base_blk = lax.dot_general(nblk, utri, (((1,), (0,)), ((), ())),
                               preferred_element_type=jnp.float32)
    base = base_blk * BLK

    slot = base + excl
    poss = []
    for oh in onehots:
        poss.append(jnp.sum(jnp.where(oh, slot, 0.0), axis=1, keepdims=True))
    pos_ref[...] = jnp.concatenate(poss, axis=1).astype(jnp.int32)

    bstart = lax.broadcasted_iota(jnp.int32, (8, NB), 1).astype(jnp.float32) * BLK
    owner = jnp.zeros((8, NB), jnp.float32)
    for e in range(E):
        owner = owner + (base[0, e] <= bstart).astype(jnp.float32)
    owner = owner - 1.0
    total_rows = jnp.sum(nblk) * BLK
    act = (bstart < total_rows).astype(jnp.float32)
    row = lax.broadcasted_iota(jnp.int32, (8, NB), 0)
    be_ref[...] = jnp.where(row == 1, act, owner).astype(jnp.int32)


def _run_router(x, router_weight, bias):
    return pl.pallas_call(
        _router_body,
        out_shape=(
            jax.ShapeDtypeStruct((T, TOPK), jnp.float32),
            jax.ShapeDtypeStruct((T, TOPK), jnp.int32),
            jax.ShapeDtypeStruct((8, NB), jnp.int32),
        ),
    )(x, router_weight, bias.reshape(1, E))


def _sc_scatter_body(x_hbm, tok_hbm, pos_hbm, xs_hbm, tokv, posv, buf, sem):
    wid = lax.axis_index("s") * NC + lax.axis_index("c")
    a0 = wid * A_W
    for j in range(NCH):
        pltpu.sync_copy(tok_hbm.at[pl.ds(a0 + j * CH, CH)], tokv)
        pltpu.sync_copy(pos_hbm.at[pl.ds(a0 + j * CH, CH)], posv)
        pltpu.async_copy(x_hbm.at[tokv], buf, sem).wait()
        pltpu.async_copy(buf, xs_hbm.at[posv], sem).wait()


def _run_sc_scatter(x, tok_flat, pos_flat):
    mesh = plsc.VectorSubcoreMesh(core_axis_name="c", subcore_axis_name="s")
    fn = functools.partial(
        pl.kernel,
        out_type=jax.ShapeDtypeStruct((PAD, H), jnp.float32),
        mesh=mesh,
        scratch_types=[
            pltpu.VMEM((CH,), jnp.int32),
            pltpu.VMEM((CH,), jnp.int32),
            pltpu.VMEM((CH, H), jnp.float32),
            pltpu.SemaphoreType.DMA,
        ],
    )(_sc_scatter_body)
    return fn(x, tok_flat, pos_flat)


def _mlp_body(be_ref, act_ref, xs_ref, guw_ref, dw_ref, o_ref):
    @pl.when(act_ref[pl.program_id(0)] > 0)
    def _():
        x = xs_ref[...].astype(jnp.bfloat16)
        guw = guw_ref[0]
        gu = lax.dot_general(x, guw, (((1,), (1,)), ((), ())),
                             preferred_element_type=jnp.float32)
        g = gu[:, :I]
        u = gu[:, I:]
        inter = (g * jax.nn.sigmoid(g) * u).astype(jnp.bfloat16)
        dw = dw_ref[0]
        o_ref[...] = lax.dot_general(inter, dw, (((1,), (1,)), ((), ())),
                                     preferred_element_type=jnp.float32)


def _run_grouped_mlp(be, act, xs, gate_up_w, down_w):
    grid_spec = pltpu.PrefetchScalarGridSpec(
        num_scalar_prefetch=2,
        grid=(NB,),
        in_specs=[
            pl.BlockSpec((BLK, H), lambda i, be, act: (i, 0)),
            pl.BlockSpec((1, 2 * I, H), lambda i, be, act: (be[i], 0, 0)),
            pl.BlockSpec((1, H, I), lambda i, be, act: (be[i], 0, 0)),
        ],
        out_specs=pl.BlockSpec((BLK, H), lambda i, be, act: (i, 0)),
    )
    return pl.pallas_call(
        _mlp_body,
        grid_spec=grid_spec,
        out_shape=jax.ShapeDtypeStruct((PAD, H), jnp.float32),
    )(be, act, xs, gate_up_w, down_w)


SBLK = 256


def _shared_body(x_ref, gw_ref, uw_ref, dw_ref, o_ref):
    x = x_ref[...]
    g = lax.dot_general(x, gw_ref[...], (((1,), (1,)), ((), ())),
                        preferred_element_type=jnp.float32)
    u = lax.dot_general(x, uw_ref[...], (((1,), (1,)), ((), ())),
                        preferred_element_type=jnp.float32)
    inter = (g * jax.nn.sigmoid(g) * u).astype(jnp.bfloat16)
    o_ref[...] = lax.dot_general(inter, dw_ref[...], (((1,), (1,)), ((), ())),
                                 preferred_element_type=jnp.float32)


def _run_shared_mlp(x, gw, uw, dw):
    return pl.pallas_call(
        _shared_body,
        grid=(T // SBLK,),
        in_specs=[
            pl.BlockSpec((SBLK, H), lambda i: (i, 0)),
            pl.BlockSpec((IS, H), lambda i: (0, 0)),
            pl.BlockSpec((IS, H), lambda i: (0, 0)),
            pl.BlockSpec((H, IS), lambda i: (0, 0)),
        ],
        out_specs=pl.BlockSpec((SBLK, H), lambda i: (i, 0)),
        out_shape=jax.ShapeDtypeStruct((T, H), jnp.float32),
    )(x, gw, uw, dw)


TCH = CH // TOPK

NVEC = H // 16


def _sc_combine_body(base_hbm, y_hbm, pos_hbm, w_hbm, out_hbm,
                     posv, wv, buf, acc, sem):
    wid = lax.axis_index("s") * NC + lax.axis_index("c")
    t0 = wid * (T // NW)
    a0 = wid * A_W
    pltpu.sync_copy(base_hbm.at[pl.ds(t0, T // NW)], acc)
    for j in range(NCH):
        pltpu.sync_copy(pos_hbm.at[pl.ds(a0 + j * CH, CH)], posv)
        pltpu.sync_copy(w_hbm.at[pl.ds(a0 + j * CH, CH)], wv.at[pl.ds(0, CH)])
        pltpu.async_copy(y_hbm.at[posv], buf, sem).wait()

        def row_body(i, _, j=j):
            a_loc = j * CH + i
            r = a_loc // TOPK
            w_r = wv[pl.ds(i, 16)][0]
            for c in range(NVEC):
                sl = pl.ds(c * 16, 16)
                acc[r, sl] = acc[r, sl] + w_r * buf[i, sl]
            return 0

        lax.fori_loop(0, CH, row_body, 0)
    pltpu.sync_copy(acc, out_hbm.at[pl.ds(t0, T // NW)])


def _run_sc_combine(shared_out, y, pos_flat, w_flat):
    mesh = plsc.VectorSubcoreMesh(core_axis_name="c", subcore_axis_name="s")
    fn = functools.partial(
        pl.kernel,
        out_type=jax.ShapeDtypeStruct((T, H), jnp.float32),
        mesh=mesh,
        scratch_types=[
            pltpu.VMEM((CH,), jnp.int32),
            pltpu.VMEM((CH + 16,), jnp.float32),
            pltpu.VMEM((CH, H), jnp.float32),
            pltpu.VMEM((T // NW, H), jnp.float32),
            pltpu.SemaphoreType.DMA,
        ],
    )(_sc_combine_body)
    return fn(shared_out, y, pos_flat, w_flat)


def kernel(hidden_states, router_weight, bias, gate_up_w, down_w,
           shared_gate_w, shared_up_w, shared_down_w):
    orig_shape = hidden_states.shape
    x = hidden_states.reshape(T, H)

    topk_w, pos, be8 = _run_router(x, router_weight, bias)
    be = be8[0]
    act = be8[1]
    pos_flat = pos.reshape(A)
    w_flat = topk_w.reshape(A)
    tok_flat = jnp.repeat(jnp.arange(T, dtype=jnp.int32), TOPK)

    x16 = x.astype(jnp.bfloat16)
    xs = _run_sc_scatter(x, tok_flat, pos_flat)
    y = _run_grouped_mlp(be, act, xs, gate_up_w.astype(jnp.bfloat16),
                         down_w.astype(jnp.bfloat16))
    shared_out = _run_shared_mlp(x16, shared_gate_w.astype(jnp.bfloat16),
                                 shared_up_w.astype(jnp.bfloat16),
                                 shared_down_w.astype(jnp.bfloat16))
    out = _run_sc_combine(shared_out, y, pos_flat, w_flat)

    out = out.reshape(orig_shape)
    return (out, out)


# speedup vs baseline: 1.3623x; 1.3623x over previous
import functools

import jax
import jax.numpy as jnp
from jax import lax
from jax.experimental import pallas as pl
from jax.experimental.pallas import tpu as pltpu
from jax.experimental.pallas import tpu_sc as plsc

H = 768
E = 128
I = 1856
IS = 3712
TOPK = 6
RSF = 2.5
T = 2048

BLK = 128
NB = 224
PAD = NB * BLK

NC = 2
NS = 16
NW = NC * NS
A = T * TOPK
A_W = A // NW
CH = 64
NCH = A_W // CH


def _router_body(x_ref, rw_ref, bias_ref, w_ref, pos_ref, be_ref):
    x = x_ref[...]
    rw = rw_ref[...]
    logits = lax.dot_general(
        x, rw, (((1,), (1,)), ((), ())),
        preferred_element_type=jnp.float32)
    scores = jax.nn.sigmoid(logits)
    sfc = scores + bias_ref[...]

    lane = lax.broadcasted_iota(jnp.int32, (T, E), 1)
    cur = sfc
    onehots = []
    ws = []
    for _ in range(TOPK):
        m = jnp.max(cur, axis=1, keepdims=True)
        idx = jnp.min(jnp.where(cur == m, lane, E), axis=1, keepdims=True)
        oh = lane == idx
        onehots.append(oh)
        ws.append(jnp.sum(jnp.where(oh, scores, 0.0), axis=1, keepdims=True))
        cur = jnp.where(oh, -1e30, cur)

    w = jnp.concatenate(ws, axis=1)
    denom = jnp.sum(w, axis=1, keepdims=True) + 1e-20
    w_ref[...] = w / denom * RSF

    hist = jnp.zeros((T, E), jnp.float32)
    for oh in onehots:
        hist = hist + oh.astype(jnp.float32)

    r128 = lax.broadcasted_iota(jnp.int32, (BLK, BLK), 0)
    c128 = lax.broadcasted_iota(jnp.int32, (BLK, BLK), 1)
    ltri = (r128 > c128).astype(jnp.float32)
    carry = jnp.zeros((1, E), jnp.float32)
    chunks = []
    for c in range(T // BLK):
        h_c = hist[c * BLK:(c + 1) * BLK, :]
        e_c = lax.dot_general(ltri, h_c, (((1,), (0,)), ((), ())),
                              preferred_element_type=jnp.float32) + carry
        chunks.append(e_c)
        carry = carry + jnp.sum(h_c, axis=0, keepdims=True)
    excl = jnp.concatenate(chunks, axis=0)
    counts = carry

    nblk = jnp.ceil(counts / BLK)
    re_ = lax.broadcasted_iota(jnp.int32, (E, E), 0)
    ce_ = lax.broadcasted_iota(jnp.int32, (E, E), 1)
    utri = (re_ < ce_).astype(jnp.float32)
    base_blk = lax.dot_general(nblk, utri, (((1,), (0,)), ((), ())),
                               preferred_element_type=jnp.float32)
    base = base_blk * BLK

    slot = base + excl
    poss = []
    for oh in onehots:
        poss.append(jnp.sum(jnp.where(oh, slot, 0.0), axis=1, keepdims=True))
    pos_ref[...] = jnp.concatenate(poss, axis=1).astype(jnp.int32)

    bstart = lax.broadcasted_iota(jnp.int32, (8, NB), 1).astype(jnp.float32) * BLK
    owner = jnp.zeros((8, NB), jnp.float32)
    for e in range(E):
        owner = owner + (base[0, e] <= bstart).astype(jnp.float32)
    owner = owner - 1.0
    total_rows = jnp.sum(nblk) * BLK
    act = (bstart < total_rows).astype(jnp.float32)
    row = lax.broadcasted_iota(jnp.int32, (8, NB), 0)
    be_ref[...] = jnp.where(row == 1, act, owner).astype(jnp.int32)


def _run_router(x, router_weight, bias):
    return pl.pallas_call(
        _router_body,
        out_shape=(
            jax.ShapeDtypeStruct((T, TOPK), jnp.float32),
            jax.ShapeDtypeStruct((T, TOPK), jnp.int32),
            jax.ShapeDtypeStruct((8, NB), jnp.int32),
        ),
    )(x, router_weight, bias.reshape(1, E))


def _sc_scatter_body(x_hbm, tok_hbm, pos_hbm, xs_hbm, tokv, posv, buf, sem):
    wid = lax.axis_index("s") * NC + lax.axis_index("c")
    a0 = wid * A_W
    for j in range(NCH):
        pltpu.sync_copy(tok_hbm.at[pl.ds(a0 + j * CH, CH)], tokv)
        pltpu.sync_copy(pos_hbm.at[pl.ds(a0 + j * CH, CH)], posv)
        pltpu.async_copy(x_hbm.at[tokv], buf, sem).wait()
        pltpu.async_copy(buf, xs_hbm.at[posv], sem).wait()


def _run_sc_scatter(x, tok_flat, pos_flat):
    mesh = plsc.VectorSubcoreMesh(core_axis_name="c", subcore_axis_name="s")
    fn = functools.partial(
        pl.kernel,
        out_type=jax.ShapeDtypeStruct((PAD, H), jnp.float32),
        mesh=mesh,
        scratch_types=[
            pltpu.VMEM((CH,), jnp.int32),
            pltpu.VMEM((CH,), jnp.int32),
            pltpu.VMEM((CH, H), jnp.float32),
            pltpu.SemaphoreType.DMA,
        ],
    )(_sc_scatter_body)
    return fn(x, tok_flat, pos_flat)


def _mlp_body(be_ref, act_ref, xs_ref, guw_ref, dw_ref, o_ref):
    @pl.when(act_ref[pl.program_id(0)] > 0)
    def _():
        x = xs_ref[...]
        guw = guw_ref[0]
        gu = lax.dot_general(x, guw, (((1,), (1,)), ((), ())),
                             preferred_element_type=jnp.float32)
        g = gu[:, :I]
        u = gu[:, I:]
        inter = g * jax.nn.sigmoid(g) * u
        dw = dw_ref[0]
        o_ref[...] = lax.dot_general(inter, dw, (((1,), (1,)), ((), ())),
                                     preferred_element_type=jnp.float32)


def _run_grouped_mlp(be, act, xs, gate_up_w, down_w):
    grid_spec = pltpu.PrefetchScalarGridSpec(
        num_scalar_prefetch=2,
        grid=(NB,),
        in_specs=[
            pl.BlockSpec((BLK, H), lambda i, be, act: (i, 0)),
            pl.BlockSpec((1, 2 * I, H), lambda i, be, act: (be[i], 0, 0)),
            pl.BlockSpec((1, H, I), lambda i, be, act: (be[i], 0, 0)),
        ],
        out_specs=pl.BlockSpec((BLK, H), lambda i, be, act: (i, 0)),
    )
    return pl.pallas_call(
        _mlp_body,
        grid_spec=grid_spec,
        out_shape=jax.ShapeDtypeStruct((PAD, H), jnp.float32),
    )(be, act, xs, gate_up_w, down_w)


SBLK = 256


def _shared_body(x_ref, gw_ref, uw_ref, dw_ref, o_ref):
    x = x_ref[...]
    g = lax.dot_general(x, gw_ref[...], (((1,), (1,)), ((), ())),
                        preferred_element_type=jnp.float32)
    u = lax.dot_general(x, uw_ref[...], (((1,), (1,)), ((), ())),
                        preferred_element_type=jnp.float32)
    inter = g * jax.nn.sigmoid(g) * u
    o_ref[...] = lax.dot_general(inter, dw_ref[...], (((1,), (1,)), ((), ())),
                                 preferred_element_type=jnp.float32)


def _run_shared_mlp(x, gw, uw, dw):
    return pl.pallas_call(
        _shared_body,
        grid=(T // SBLK,),
        in_specs=[
            pl.BlockSpec((SBLK, H), lambda i: (i, 0)),
            pl.BlockSpec((IS, H), lambda i: (0, 0)),
            pl.BlockSpec((IS, H), lambda i: (0, 0)),
            pl.BlockSpec((H, IS), lambda i: (0, 0)),
        ],
        out_specs=pl.BlockSpec((SBLK, H), lambda i: (i, 0)),
        out_shape=jax.ShapeDtypeStruct((T, H), jnp.float32),
    )(x, gw, uw, dw)


TCH = CH // TOPK

NVEC = H // 16


def _sc_combine_body(base_hbm, y_hbm, pos_hbm, w_hbm, out_hbm,
                     posv, wv, buf, acc, sem):
    wid = lax.axis_index("s") * NC + lax.axis_index("c")
    t0 = wid * (T // NW)
    a0 = wid * A_W
    pltpu.sync_copy(base_hbm.at[pl.ds(t0, T // NW)], acc)
    for j in range(NCH):
        pltpu.sync_copy(pos_hbm.at[pl.ds(a0 + j * CH, CH)], posv)
        pltpu.sync_copy(w_hbm.at[pl.ds(a0 + j * CH, CH)], wv.at[pl.ds(0, CH)])
        pltpu.async_copy(y_hbm.at[posv], buf, sem).wait()

        def row_body(i, _, j=j):
            a_loc = j * CH + i
            r = a_loc // TOPK
            w_r = wv[pl.ds(i, 16)][0]
            for c in range(NVEC):
                sl = pl.ds(c * 16, 16)
                acc[r, sl] = acc[r, sl] + w_r * buf[i, sl]
            return 0

        lax.fori_loop(0, CH, row_body, 0)
    pltpu.sync_copy(acc, out_hbm.at[pl.ds(t0, T // NW)])


def _run_sc_combine(shared_out, y, pos_flat, w_flat):
    mesh = plsc.VectorSubcoreMesh(core_axis_name="c", subcore_axis_name="s")
    fn = functools.partial(
        pl.kernel,
        out_type=jax.ShapeDtypeStruct((T, H), jnp.float32),
        mesh=mesh,
        scratch_types=[
            pltpu.VMEM((CH,), jnp.int32),
            pltpu.VMEM((CH + 16,), jnp.float32),
            pltpu.VMEM((CH, H), jnp.float32),
            pltpu.VMEM((T // NW, H), jnp.float32),
            pltpu.SemaphoreType.DMA,
        ],
    )(_sc_combine_body)
    return fn(shared_out, y, pos_flat, w_flat)


def kernel(hidden_states, router_weight, bias, gate_up_w, down_w,
           shared_gate_w, shared_up_w, shared_down_w):
    orig_shape = hidden_states.shape
    x = hidden_states.reshape(T, H)

    topk_w, pos, be8 = _run_router(x, router_weight, bias)
    be = be8[0]
    act = be8[1]
    pos_flat = pos.reshape(A)
    w_flat = topk_w.reshape(A)
    tok_flat = jnp.repeat(jnp.arange(T, dtype=jnp.int32), TOPK)

    xs = _run_sc_scatter(x, tok_flat, pos_flat)
    y = _run_grouped_mlp(be, act, xs, gate_up_w, down_w)
    shared_out = _run_shared_mlp(x, shared_gate_w, shared_up_w, shared_down_w)
    out = _run_sc_combine(shared_out, y, pos_flat, w_flat)

    out = out.reshape(orig_shape)
    return (out, out)


# speedup vs baseline: 1.4494x; 1.0639x over previous
import functools

import jax
import jax.numpy as jnp
from jax import lax
from jax.experimental import pallas as pl
from jax.experimental.pallas import tpu as pltpu
from jax.experimental.pallas import tpu_sc as plsc

H = 768
E = 128
I = 1856
IS = 3712
TOPK = 6
RSF = 2.5
T = 2048

BLK = 128
NB = 224
PAD = NB * BLK

NC = 2
NS = 16
NW = NC * NS
A = T * TOPK
A_W = A // NW
CH = 64
NCH = A_W // CH


def _router_body(x_ref, rw_ref, bias_ref, w_ref, pos_ref, be_ref):
    x = x_ref[...]
    rw = rw_ref[...]
    logits = lax.dot_general(
        x, rw, (((1,), (1,)), ((), ())),
        preferred_element_type=jnp.float32)
    scores = jax.nn.sigmoid(logits)
    sfc = scores + bias_ref[...]

    lane = lax.broadcasted_iota(jnp.int32, (T, E), 1)
    cur = sfc
    onehots = []
    ws = []
    for _ in range(TOPK):
        m = jnp.max(cur, axis=1, keepdims=True)
        idx = jnp.min(jnp.where(cur == m, lane, E), axis=1, keepdims=True)
        oh = lane == idx
        onehots.append(oh)
        ws.append(jnp.sum(jnp.where(oh, scores, 0.0), axis=1, keepdims=True))
        cur = jnp.where(oh, -1e30, cur)

    w = jnp.concatenate(ws, axis=1)
    denom = jnp.sum(w, axis=1, keepdims=True) + 1e-20
    w_ref[...] = w / denom * RSF

    hist = jnp.zeros((T, E), jnp.float32)
    for oh in onehots:
        hist = hist + oh.astype(jnp.float32)

    r128 = lax.broadcasted_iota(jnp.int32, (BLK, BLK), 0)
    c128 = lax.broadcasted_iota(jnp.int32, (BLK, BLK), 1)
    ltri = (r128 > c128).astype(jnp.float32)
    carry = jnp.zeros((1, E), jnp.float32)
    chunks = []
    for c in range(T // BLK):
        h_c = hist[c * BLK:(c + 1) * BLK, :]
        e_c = lax.dot_general(ltri, h_c, (((1,), (0,)), ((), ())),
                              preferred_element_type=jnp.float32) + carry
        chunks.append(e_c)
        carry = carry + jnp.sum(h_c, axis=0, keepdims=True)
    excl = jnp.concatenate(chunks, axis=0)
    counts = carry

    nblk = jnp.ceil(counts / BLK)
    re_ = lax.broadcasted_iota(jnp.int32, (E, E), 0)
    ce_ = lax.broadcasted_iota(jnp.int32, (E, E), 1)
    utri = (re_ < ce_).astype(jnp.float32)
    base_blk = lax.dot_general(nblk, utri, (((1,), (0,)), ((), ())),
                               preferred_element_type=jnp.float32)
    base = base_blk * BLK

    slot = base + excl
    poss = []
    for oh in onehots:
        poss.append(jnp.sum(jnp.where(oh, slot, 0.0), axis=1, keepdims=True))
    pos_ref[...] = jnp.concatenate(poss, axis=1).astype(jnp.int32)

    bstart = lax.broadcasted_iota(jnp.int32, (8, NB), 1).astype(jnp.float32) * BLK
    owner = jnp.zeros((8, NB), jnp.float32)
    for e in range(E):
        owner = owner + (base[0, e] <= bstart).astype(jnp.float32)
    owner = owner - 1.0
    total_rows = jnp.sum(nblk) * BLK
    act = (bstart < total_rows).astype(jnp.float32)
    row = lax.broadcasted_iota(jnp.int32, (8, NB), 0)
    be_ref[...] = jnp.where(row == 1, act, owner).astype(jnp.int32)


def _run_router(x, router_weight, bias):
    return pl.pallas_call(
        _router_body,
        out_shape=(
            jax.ShapeDtypeStruct((T, TOPK), jnp.float32),
            jax.ShapeDtypeStruct((T, TOPK), jnp.int32),
            jax.ShapeDtypeStruct((8, NB), jnp.int32),
        ),
    )(x, router_weight, bias.reshape(1, E))


def _sc_scatter_body(x_hbm, tok_hbm, pos_hbm, xs_hbm,
                     tokv0, tokv1, posv0, posv1, buf0, buf1,
                     sg0, sg1, ss0, ss1):
    wid = lax.axis_index("s") * NC + lax.axis_index("c")
    a0 = wid * A_W
    tokv = (tokv0, tokv1)
    posv = (posv0, posv1)
    bufs = (buf0, buf1)
    gs = (sg0, sg1)
    ss = (ss0, ss1)
    scatters = [None, None]
    pltpu.sync_copy(tok_hbm.at[pl.ds(a0, CH)], tokv0)
    pltpu.sync_copy(pos_hbm.at[pl.ds(a0, CH)], posv0)
    g_prev = pltpu.async_copy(x_hbm.at[tokv0], buf0, sg0)
    for j in range(NCH):
        b = j % 2
        nb = (j + 1) % 2
        if j + 1 < NCH:
            if scatters[nb] is not None:
                scatters[nb].wait()
            pltpu.sync_copy(tok_hbm.at[pl.ds(a0 + (j + 1) * CH, CH)], tokv[nb])
            pltpu.sync_copy(pos_hbm.at[pl.ds(a0 + (j + 1) * CH, CH)], posv[nb])
            g_next = pltpu.async_copy(x_hbm.at[tokv[nb]], bufs[nb], gs[nb])
        g_prev.wait()
        scatters[b] = pltpu.async_copy(bufs[b], xs_hbm.at[posv[b]], ss[b])
        if j + 1 < NCH:
            g_prev = g_next
    scatters[0].wait()
    scatters[1].wait()


def _run_sc_scatter(x, tok_flat, pos_flat):
    mesh = plsc.VectorSubcoreMesh(core_axis_name="c", subcore_axis_name="s")
    fn = functools.partial(
        pl.kernel,
        out_type=jax.ShapeDtypeStruct((PAD, H), jnp.float32),
        mesh=mesh,
        scratch_types=[
            pltpu.VMEM((CH,), jnp.int32),
            pltpu.VMEM((CH,), jnp.int32),
            pltpu.VMEM((CH,), jnp.int32),
            pltpu.VMEM((CH,), jnp.int32),
            pltpu.VMEM((CH, H), jnp.float32),
            pltpu.VMEM((CH, H), jnp.float32),
            pltpu.SemaphoreType.DMA,
            pltpu.SemaphoreType.DMA,
            pltpu.SemaphoreType.DMA,
            pltpu.SemaphoreType.DMA,
        ],
    )(_sc_scatter_body)
    return fn(x, tok_flat, pos_flat)


def _mlp_body(be_ref, act_ref, xs_ref, guw_ref, dw_ref, o_ref):
    @pl.when(act_ref[pl.program_id(0)] > 0)
    def _():
        x = xs_ref[...]
        guw = guw_ref[0]
        gu = lax.dot_general(x, guw, (((1,), (1,)), ((), ())),
                             preferred_element_type=jnp.float32)
        g = gu[:, :I]
        u = gu[:, I:]
        inter = g * jax.nn.sigmoid(g) * u
        dw = dw_ref[0]
        o_ref[...] = lax.dot_general(inter, dw, (((1,), (1,)), ((), ())),
                                     preferred_element_type=jnp.float32)


def _run_grouped_mlp(be, act, xs, gate_up_w, down_w):
    grid_spec = pltpu.PrefetchScalarGridSpec(
        num_scalar_prefetch=2,
        grid=(NB,),
        in_specs=[
            pl.BlockSpec((BLK, H), lambda i, be, act: (i, 0)),
            pl.BlockSpec((1, 2 * I, H), lambda i, be, act: (be[i], 0, 0)),
            pl.BlockSpec((1, H, I), lambda i, be, act: (be[i], 0, 0)),
        ],
        out_specs=pl.BlockSpec((BLK, H), lambda i, be, act: (i, 0)),
    )
    return pl.pallas_call(
        _mlp_body,
        grid_spec=grid_spec,
        out_shape=jax.ShapeDtypeStruct((PAD, H), jnp.float32),
    )(be, act, xs, gate_up_w, down_w)


SBLK = 256


def _shared_body(x_ref, gw_ref, uw_ref, dw_ref, o_ref):
    x = x_ref[...]
    g = lax.dot_general(x, gw_ref[...], (((1,), (1,)), ((), ())),
                        preferred_element_type=jnp.float32)
    u = lax.dot_general(x, uw_ref[...], (((1,), (1,)), ((), ())),
                        preferred_element_type=jnp.float32)
    inter = g * jax.nn.sigmoid(g) * u
    o_ref[...] = lax.dot_general(inter, dw_ref[...], (((1,), (1,)), ((), ())),
                                 preferred_element_type=jnp.float32)


def _run_shared_mlp(x, gw, uw, dw):
    return pl.pallas_call(
        _shared_body,
        grid=(T // SBLK,),
        in_specs=[
            pl.BlockSpec((SBLK, H), lambda i: (i, 0)),
            pl.BlockSpec((IS, H), lambda i: (0, 0)),
            pl.BlockSpec((IS, H), lambda i: (0, 0)),
            pl.BlockSpec((H, IS), lambda i: (0, 0)),
        ],
        out_specs=pl.BlockSpec((SBLK, H), lambda i: (i, 0)),
        out_shape=jax.ShapeDtypeStruct((T, H), jnp.float32),
    )(x, gw, uw, dw)


NVEC = H // 16
GT = 8
GC = GT * TOPK
NG = (T // NW) // GT


def _sc_combine_body(base_hbm, y_hbm, pos_hbm, w_hbm, out_hbm,
                     posv, wv, buf0, buf1, acc, semb, sem0, sem1):
    wid = lax.axis_index("s") * NC + lax.axis_index("c")
    t0 = wid * (T // NW)
    a0 = wid * A_W
    pltpu.sync_copy(pos_hbm.at[pl.ds(a0, A_W)], posv)
    pltpu.sync_copy(w_hbm.at[pl.ds(a0, A_W)], wv.at[pl.ds(0, A_W)])
    cb = pltpu.async_copy(base_hbm.at[pl.ds(t0, T // NW)], acc, semb)
    bufs = (buf0, buf1)
    sems = (sem0, sem1)
    handles = [None, None]
    handles[0] = pltpu.async_copy(y_hbm.at[posv.at[pl.ds(0, GC)]], buf0, sem0)
    cb.wait()
    for g in range(NG):
        b = g % 2
        nb = (g + 1) % 2
        if g + 1 < NG:
            handles[nb] = pltpu.async_copy(
                y_hbm.at[posv.at[pl.ds((g + 1) * GC, GC)]], bufs[nb], sems[nb])
        handles[b].wait()
        buf = bufs[b]
        wsc = []
        for t in range(GT):
            wt = wv[pl.ds(g * GC + t * TOPK, 16)]
            wsc.append([wt[k] for k in range(TOPK)])

        def cbody(c, _, g=g, buf=buf, wsc=wsc):
            sl = pl.ds(c * 16, 16)
            for t in range(GT):
                av = acc[g * GT + t, sl]
                for k in range(TOPK):
                    av = av + wsc[t][k] * buf[t * TOPK + k, sl]
                acc[g * GT + t, sl] = av
            return 0

        lax.fori_loop(0, NVEC, cbody, 0)
    pltpu.sync_copy(acc, out_hbm.at[pl.ds(t0, T // NW)])


def _run_sc_combine(shared_out, y, pos_flat, w_flat):
    mesh = plsc.VectorSubcoreMesh(core_axis_name="c", subcore_axis_name="s")
    fn = functools.partial(
        pl.kernel,
        out_type=jax.ShapeDtypeStruct((T, H), jnp.float32),
        mesh=mesh,
        scratch_types=[
            pltpu.VMEM((A_W,), jnp.int32),
            pltpu.VMEM((A_W + 16,), jnp.float32),
            pltpu.VMEM((GC, H), jnp.float32),
            pltpu.VMEM((GC, H), jnp.float32),
            pltpu.VMEM((T // NW, H), jnp.float32),
            pltpu.SemaphoreType.DMA,
            pltpu.SemaphoreType.DMA,
            pltpu.SemaphoreType.DMA,
        ],
    )(_sc_combine_body)
    return fn(shared_out, y, pos_flat, w_flat)


def kernel(hidden_states, router_weight, bias, gate_up_w, down_w,
           shared_gate_w, shared_up_w, shared_down_w):
    orig_shape = hidden_states.shape
    x = hidden_states.reshape(T, H)

    topk_w, pos, be8 = _run_router(x, router_weight, bias)
    be = be8[0]
    act = be8[1]
    pos_flat = pos.reshape(A)
    w_flat = topk_w.reshape(A)
    tok_flat = jnp.repeat(jnp.arange(T, dtype=jnp.int32), TOPK)

    shared_out = _run_shared_mlp(x, shared_gate_w, shared_up_w, shared_down_w)
    xs = _run_sc_scatter(x, tok_flat, pos_flat)
    y = _run_grouped_mlp(be, act, xs, gate_up_w, down_w)
    out = _run_sc_combine(shared_out, y, pos_flat, w_flat)

    out = out.reshape(orig_shape)
    return (out, out)


# speedup vs baseline: 1.6212x; 1.1185x over previous
import functools

import jax
import jax.numpy as jnp
from jax import lax
from jax.experimental import pallas as pl
from jax.experimental.pallas import tpu as pltpu
from jax.experimental.pallas import tpu_sc as plsc

H = 768
E = 128
I = 1856
IS = 3712
TOPK = 6
RSF = 2.5
T = 2048

BLK = 128
NB = 224
PAD = NB * BLK

NC = 2
NS = 16
NW = NC * NS
A = T * TOPK
A_W = A // NW
CH = 64
NCH = A_W // CH


def _router_body(x_ref, rw_ref, bias_ref, w_ref, pos_ref, be_ref):
    x = x_ref[...]
    rw = rw_ref[...]
    logits = lax.dot_general(
        x, rw, (((1,), (1,)), ((), ())),
        preferred_element_type=jnp.float32)
    scores = jax.nn.sigmoid(logits)
    sfc = scores + bias_ref[...]

    lane = lax.broadcasted_iota(jnp.int32, (T, E), 1)
    cur = sfc
    onehots = []
    ws = []
    for _ in range(TOPK):
        m = jnp.max(cur, axis=1, keepdims=True)
        idx = jnp.min(jnp.where(cur == m, lane, E), axis=1, keepdims=True)
        oh = lane == idx
        onehots.append(oh)
        ws.append(jnp.sum(jnp.where(oh, scores, 0.0), axis=1, keepdims=True))
        cur = jnp.where(oh, -1e30, cur)

    w = jnp.concatenate(ws, axis=1)
    denom = jnp.sum(w, axis=1, keepdims=True) + 1e-20
    w_ref[...] = w / denom * RSF

    hist = jnp.zeros((T, E), jnp.float32)
    for oh in onehots:
        hist = hist + oh.astype(jnp.float32)

    r128 = lax.broadcasted_iota(jnp.int32, (BLK, BLK), 0)
    c128 = lax.broadcasted_iota(jnp.int32, (BLK, BLK), 1)
    ltri = (r128 > c128).astype(jnp.float32)
    carry = jnp.zeros((1, E), jnp.float32)
    chunks = []
    for c in range(T // BLK):
        h_c = hist[c * BLK:(c + 1) * BLK, :]
        e_c = lax.dot_general(ltri, h_c, (((1,), (0,)), ((), ())),
                              preferred_element_type=jnp.float32) + carry
        chunks.append(e_c)
        carry = carry + jnp.sum(h_c, axis=0, keepdims=True)
    excl = jnp.concatenate(chunks, axis=0)
    counts = carry

    nblk = jnp.ceil(counts / BLK)
    re_ = lax.broadcasted_iota(jnp.int32, (E, E), 0)
    ce_ = lax.broadcasted_iota(jnp.int32, (E, E), 1)
    utri = (re_ < ce_).astype(jnp.float32)
    base_blk = lax.dot_general(nblk, utri, (((1,), (0,)), ((), ())),
                               preferred_element_type=jnp.float32)
    base = base_blk * BLK

    slot = base + excl
    poss = []
    for oh in onehots:
        poss.append(jnp.sum(jnp.where(oh, slot, 0.0), axis=1, keepdims=True))
    pos_ref[...] = jnp.concatenate(poss, axis=1).astype(jnp.int32)

    bstart = lax.broadcasted_iota(jnp.int32, (8, NB), 1).astype(jnp.float32) * BLK
    owner = jnp.zeros((8, NB), jnp.float32)
    for e in range(E):
        owner = owner + (base[0, e] <= bstart).astype(jnp.float32)
    owner = owner - 1.0
    total_rows = jnp.sum(nblk) * BLK
    act = (bstart < total_rows).astype(jnp.float32)
    row = lax.broadcasted_iota(jnp.int32, (8, NB), 0)
    be_ref[...] = jnp.where(row == 1, act, owner).astype(jnp.int32)


def _run_router(x, router_weight, bias):
    return pl.pallas_call(
        _router_body,
        out_shape=(
            jax.ShapeDtypeStruct((T, TOPK), jnp.float32),
            jax.ShapeDtypeStruct((T, TOPK), jnp.int32),
            jax.ShapeDtypeStruct((8, NB), jnp.int32),
        ),
    )(x, router_weight, bias.reshape(1, E))


def _sc_scatter_body(x_hbm, tok_hbm, pos_hbm, xs_hbm,
                     tokv0, tokv1, posv0, posv1, buf0, buf1,
                     sg0, sg1, ss0, ss1):
    wid = lax.axis_index("s") * NC + lax.axis_index("c")
    a0 = wid * A_W
    tokv = (tokv0, tokv1)
    posv = (posv0, posv1)
    bufs = (buf0, buf1)
    gs = (sg0, sg1)
    ss = (ss0, ss1)
    scatters = [None, None]
    pltpu.sync_copy(tok_hbm.at[pl.ds(a0, CH)], tokv0)
    pltpu.sync_copy(pos_hbm.at[pl.ds(a0, CH)], posv0)
    g_prev = pltpu.async_copy(x_hbm.at[tokv0], buf0, sg0)
    for j in range(NCH):
        b = j % 2
        nb = (j + 1) % 2
        if j + 1 < NCH:
            if scatters[nb] is not None:
                scatters[nb].wait()
            pltpu.sync_copy(tok_hbm.at[pl.ds(a0 + (j + 1) * CH, CH)], tokv[nb])
            pltpu.sync_copy(pos_hbm.at[pl.ds(a0 + (j + 1) * CH, CH)], posv[nb])
            g_next = pltpu.async_copy(x_hbm.at[tokv[nb]], bufs[nb], gs[nb])
        g_prev.wait()
        scatters[b] = pltpu.async_copy(bufs[b], xs_hbm.at[posv[b]], ss[b])
        if j + 1 < NCH:
            g_prev = g_next
    scatters[0].wait()
    scatters[1].wait()


def _run_sc_scatter(x, tok_flat, pos_flat):
    mesh = plsc.VectorSubcoreMesh(core_axis_name="c", subcore_axis_name="s")
    fn = functools.partial(
        pl.kernel,
        out_type=jax.ShapeDtypeStruct((PAD, H), jnp.float32),
        mesh=mesh,
        scratch_types=[
            pltpu.VMEM((CH,), jnp.int32),
            pltpu.VMEM((CH,), jnp.int32),
            pltpu.VMEM((CH,), jnp.int32),
            pltpu.VMEM((CH,), jnp.int32),
            pltpu.VMEM((CH, H), jnp.float32),
            pltpu.VMEM((CH, H), jnp.float32),
            pltpu.SemaphoreType.DMA,
            pltpu.SemaphoreType.DMA,
            pltpu.SemaphoreType.DMA,
            pltpu.SemaphoreType.DMA,
        ],
    )(_sc_scatter_body)
    return fn(x, tok_flat, pos_flat)


HH = H // 2


def _mlp_body(be_ref, act_ref, xs_ref, g_ref, u_ref, dwa_ref, dwb_ref, o_ref):
    @pl.when(act_ref[pl.program_id(0)] > 0)
    def _():
        x = xs_ref[...]
        g = lax.dot_general(x, g_ref[0, 0], (((1,), (1,)), ((), ())),
                            preferred_element_type=jnp.float32)
        u = lax.dot_general(x, u_ref[0, 0], (((1,), (1,)), ((), ())),
                            preferred_element_type=jnp.float32)
        inter = g * jax.nn.sigmoid(g) * u
        o_ref[:, :HH] = lax.dot_general(inter, dwa_ref[0, 0],
                                        (((1,), (1,)), ((), ())),
                                        preferred_element_type=jnp.float32)
        o_ref[:, HH:] = lax.dot_general(inter, dwb_ref[0, 0],
                                        (((1,), (1,)), ((), ())),
                                        preferred_element_type=jnp.float32)


def _run_grouped_mlp(be, act, xs, gate_up_w, down_w):
    guw4 = gate_up_w.reshape(E, 2, I, H)
    dw4 = down_w.reshape(E, 2, HH, I)
    grid_spec = pltpu.PrefetchScalarGridSpec(
        num_scalar_prefetch=2,
        grid=(NB,),
        in_specs=[
            pl.BlockSpec((BLK, H), lambda i, be, act: (i, 0)),
            pl.BlockSpec((1, 1, I, H), lambda i, be, act: (be[i], 0, 0, 0)),
            pl.BlockSpec((1, 1, I, H), lambda i, be, act: (be[i], 1, 0, 0)),
            pl.BlockSpec((1, 1, HH, I), lambda i, be, act: (be[i], 0, 0, 0)),
            pl.BlockSpec((1, 1, HH, I), lambda i, be, act: (be[i], 1, 0, 0)),
        ],
        out_specs=pl.BlockSpec((BLK, H), lambda i, be, act: (i, 0)),
    )
    return pl.pallas_call(
        _mlp_body,
        grid_spec=grid_spec,
        out_shape=jax.ShapeDtypeStruct((PAD, H), jnp.float32),
    )(be, act, xs, guw4, guw4, dw4, dw4)


SBLK = 256


def _shared_body(x_ref, gw_ref, uw_ref, dw_ref, o_ref):
    x = x_ref[...]
    g = lax.dot_general(x, gw_ref[...], (((1,), (1,)), ((), ())),
                        preferred_element_type=jnp.float32)
    u = lax.dot_general(x, uw_ref[...], (((1,), (1,)), ((), ())),
                        preferred_element_type=jnp.float32)
    inter = g * jax.nn.sigmoid(g) * u
    o_ref[...] = lax.dot_general(inter, dw_ref[...], (((1,), (1,)), ((), ())),
                                 preferred_element_type=jnp.float32)


def _run_shared_mlp(x, gw, uw, dw):
    return pl.pallas_call(
        _shared_body,
        grid=(T // SBLK,),
        in_specs=[
            pl.BlockSpec((SBLK, H), lambda i: (i, 0)),
            pl.BlockSpec((IS, H), lambda i: (0, 0)),
            pl.BlockSpec((IS, H), lambda i: (0, 0)),
            pl.BlockSpec((H, IS), lambda i: (0, 0)),
        ],
        out_specs=pl.BlockSpec((SBLK, H), lambda i: (i, 0)),
        out_shape=jax.ShapeDtypeStruct((T, H), jnp.float32),
    )(x, gw, uw, dw)


NVEC = H // 16
GT = 8
GC = GT * TOPK
NG = (T // NW) // GT


def _sc_combine_body(base_hbm, y_hbm, pos_hbm, w_hbm, out_hbm,
                     posv, wv, buf0, buf1, acc, semb, sem0, sem1):
    wid = lax.axis_index("s") * NC + lax.axis_index("c")
    t0 = wid * (T // NW)
    a0 = wid * A_W
    pltpu.sync_copy(pos_hbm.at[pl.ds(a0, A_W)], posv)
    pltpu.sync_copy(w_hbm.at[pl.ds(a0, A_W)], wv.at[pl.ds(0, A_W)])
    cb = pltpu.async_copy(base_hbm.at[pl.ds(t0, T // NW)], acc, semb)
    bufs = (buf0, buf1)
    sems = (sem0, sem1)
    handles = [None, None]
    handles[0] = pltpu.async_copy(y_hbm.at[posv.at[pl.ds(0, GC)]], buf0, sem0)
    cb.wait()
    for g in range(NG):
        b = g % 2
        nb = (g + 1) % 2
        if g + 1 < NG:
            handles[nb] = pltpu.async_copy(
                y_hbm.at[posv.at[pl.ds((g + 1) * GC, GC)]], bufs[nb], sems[nb])
        handles[b].wait()
        buf = bufs[b]
        wsc = []
        for t in range(GT):
            wt = wv[pl.ds(g * GC + t * TOPK, 16)]
            wsc.append([wt[k] for k in range(TOPK)])

        def cbody(c, _, g=g, buf=buf, wsc=wsc):
            sl = pl.ds(c * 16, 16)
            for t in range(GT):
                av = acc[g * GT + t, sl]
                for k in range(TOPK):
                    av = av + wsc[t][k] * buf[t * TOPK + k, sl]
                acc[g * GT + t, sl] = av
            return 0

        lax.fori_loop(0, NVEC, cbody, 0)
    pltpu.sync_copy(acc, out_hbm.at[pl.ds(t0, T // NW)])


def _run_sc_combine(shared_out, y, pos_flat, w_flat):
    mesh = plsc.VectorSubcoreMesh(core_axis_name="c", subcore_axis_name="s")
    fn = functools.partial(
        pl.kernel,
        out_type=jax.ShapeDtypeStruct((T, H), jnp.float32),
        mesh=mesh,
        scratch_types=[
            pltpu.VMEM((A_W,), jnp.int32),
            pltpu.VMEM((A_W + 16,), jnp.float32),
            pltpu.VMEM((GC, H), jnp.float32),
            pltpu.VMEM((GC, H), jnp.float32),
            pltpu.VMEM((T // NW, H), jnp.float32),
            pltpu.SemaphoreType.DMA,
            pltpu.SemaphoreType.DMA,
            pltpu.SemaphoreType.DMA,
        ],
    )(_sc_combine_body)
    return fn(shared_out, y, pos_flat, w_flat)


def kernel(hidden_states, router_weight, bias, gate_up_w, down_w,
           shared_gate_w, shared_up_w, shared_down_w):
    orig_shape = hidden_states.shape
    x = hidden_states.reshape(T, H)

    topk_w, pos, be8 = _run_router(x, router_weight, bias)
    be = be8[0]
    act = be8[1]
    pos_flat = pos.reshape(A)
    w_flat = topk_w.reshape(A)
    tok_flat = jnp.repeat(jnp.arange(T, dtype=jnp.int32), TOPK)

    shared_out = _run_shared_mlp(x, shared_gate_w, shared_up_w, shared_down_w)
    xs = _run_sc_scatter(x, tok_flat, pos_flat)
    y = _run_grouped_mlp(be, act, xs, gate_up_w, down_w)
    out = _run_sc_combine(shared_out, y, pos_flat, w_flat)

    out = out.reshape(orig_shape)
    return (out, out)


# speedup vs baseline: 1.6216x; 1.0002x over previous
import functools

import jax
import jax.numpy as jnp
from jax import lax
from jax.experimental import pallas as pl
from jax.experimental.pallas import tpu as pltpu
from jax.experimental.pallas import tpu_sc as plsc

H = 768
E = 128
I = 1856
IS = 3712
TOPK = 6
RSF = 2.5
T = 2048

BLK = 128
NB = 224
PAD = NB * BLK

NC = 2
NS = 16
NW = NC * NS
A = T * TOPK
A_W = A // NW
CH = 64
NCH = A_W // CH


def _router_body(x_ref, rw_ref, bias_ref, w_ref, pos_ref, be_ref):
    x = x_ref[...]
    rw = rw_ref[...]
    logits = lax.dot_general(
        x, rw, (((1,), (1,)), ((), ())),
        preferred_element_type=jnp.float32)
    scores = jax.nn.sigmoid(logits)
    sfc = scores + bias_ref[...]

    lane = lax.broadcasted_iota(jnp.int32, (T, E), 1)
    cur = sfc
    onehots = []
    ws = []
    for _ in range(TOPK):
        m = jnp.max(cur, axis=1, keepdims=True)
        idx = jnp.min(jnp.where(cur == m, lane, E), axis=1, keepdims=True)
        oh = lane == idx
        onehots.append(oh)
        ws.append(jnp.sum(jnp.where(oh, scores, 0.0), axis=1, keepdims=True))
        cur = jnp.where(oh, -1e30, cur)

    w = jnp.concatenate(ws, axis=1)
    denom = jnp.sum(w, axis=1, keepdims=True) + 1e-20
    w_ref[...] = w / denom * RSF

    hist = jnp.zeros((T, E), jnp.float32)
    for oh in onehots:
        hist = hist + oh.astype(jnp.float32)

    r128 = lax.broadcasted_iota(jnp.int32, (BLK, BLK), 0)
    c128 = lax.broadcasted_iota(jnp.int32, (BLK, BLK), 1)
    ltri = (r128 > c128).astype(jnp.float32)
    carry = jnp.zeros((1, E), jnp.float32)
    chunks = []
    for c in range(T // BLK):
        h_c = hist[c * BLK:(c + 1) * BLK, :]
        e_c = lax.dot_general(ltri, h_c, (((1,), (0,)), ((), ())),
                              preferred_element_type=jnp.float32) + carry
        chunks.append(e_c)
        carry = carry + jnp.sum(h_c, axis=0, keepdims=True)
    excl = jnp.concatenate(chunks, axis=0)
    counts = carry

    nblk = jnp.ceil(counts / BLK)
    re_ = lax.broadcasted_iota(jnp.int32, (E, E), 0)
    ce_ = lax.broadcasted_iota(jnp.int32, (E, E), 1)
    utri = (re_ < ce_).astype(jnp.float32)
    base_blk = lax.dot_general(nblk, utri, (((1,), (0,)), ((), ())),
                               preferred_element_type=jnp.float32)
    base = base_blk * BLK

    slot = base + excl
    poss = []
    for oh in onehots:
        poss.append(jnp.sum(jnp.where(oh, slot, 0.0), axis=1, keepdims=True))
    pos_ref[...] = jnp.concatenate(poss, axis=1).astype(jnp.int32)

    bstart = lax.broadcasted_iota(jnp.int32, (8, NB), 1).astype(jnp.float32) * BLK
    owner = jnp.zeros((8, NB), jnp.float32)
    for e in range(E):
        owner = owner + (base[0, e] <= bstart).astype(jnp.float32)
    owner = owner - 1.0
    total_rows = jnp.sum(nblk) * BLK
    act = (bstart < total_rows).astype(jnp.float32)
    row = lax.broadcasted_iota(jnp.int32, (8, NB), 0)
    be_ref[...] = jnp.where(row == 1, act, owner).astype(jnp.int32)


def _run_router(x, router_weight, bias):
    return pl.pallas_call(
        _router_body,
        out_shape=(
            jax.ShapeDtypeStruct((T, TOPK), jnp.float32),
            jax.ShapeDtypeStruct((T, TOPK), jnp.int32),
            jax.ShapeDtypeStruct((8, NB), jnp.int32),
        ),
    )(x, router_weight, bias.reshape(1, E))


def _sc_scatter_body(x_hbm, tok_hbm, pos_hbm, xs_hbm,
                     tokv0, tokv1, posv0, posv1, buf0, buf1,
                     sg0, sg1, ss0, ss1):
    wid = lax.axis_index("s") * NC + lax.axis_index("c")
    a0 = wid * A_W
    tokv = (tokv0, tokv1)
    posv = (posv0, posv1)
    bufs = (buf0, buf1)
    gs = (sg0, sg1)
    ss = (ss0, ss1)
    scatters = [None, None]
    pltpu.sync_copy(tok_hbm.at[pl.ds(a0, CH)], tokv0)
    pltpu.sync_copy(pos_hbm.at[pl.ds(a0, CH)], posv0)
    g_prev = pltpu.async_copy(x_hbm.at[tokv0], buf0, sg0)
    for j in range(NCH):
        b = j % 2
        nb = (j + 1) % 2
        if j + 1 < NCH:
            if scatters[nb] is not None:
                scatters[nb].wait()
            pltpu.sync_copy(tok_hbm.at[pl.ds(a0 + (j + 1) * CH, CH)], tokv[nb])
            pltpu.sync_copy(pos_hbm.at[pl.ds(a0 + (j + 1) * CH, CH)], posv[nb])
            g_next = pltpu.async_copy(x_hbm.at[tokv[nb]], bufs[nb], gs[nb])
        g_prev.wait()
        scatters[b] = pltpu.async_copy(bufs[b], xs_hbm.at[posv[b]], ss[b])
        if j + 1 < NCH:
            g_prev = g_next
    scatters[0].wait()
    scatters[1].wait()


def _run_sc_scatter(x, tok_flat, pos_flat):
    mesh = plsc.VectorSubcoreMesh(core_axis_name="c", subcore_axis_name="s")
    fn = functools.partial(
        pl.kernel,
        out_type=jax.ShapeDtypeStruct((PAD, H), jnp.float32),
        mesh=mesh,
        scratch_types=[
            pltpu.VMEM((CH,), jnp.int32),
            pltpu.VMEM((CH,), jnp.int32),
            pltpu.VMEM((CH,), jnp.int32),
            pltpu.VMEM((CH,), jnp.int32),
            pltpu.VMEM((CH, H), jnp.float32),
            pltpu.VMEM((CH, H), jnp.float32),
            pltpu.SemaphoreType.DMA,
            pltpu.SemaphoreType.DMA,
            pltpu.SemaphoreType.DMA,
            pltpu.SemaphoreType.DMA,
        ],
    )(_sc_scatter_body)
    return fn(x, tok_flat, pos_flat)


IH = I // 2
HQ = H // 4


def _mlp_body(be_ref, act_ref, xs_ref, g0_ref, g1_ref, u0_ref, u1_ref,
              d0_ref, d1_ref, d2_ref, d3_ref, o_ref):
    @pl.when(act_ref[pl.program_id(0)] > 0)
    def _():
        x = xs_ref[...]

        def mm(a, b):
            return lax.dot_general(a, b, (((1,), (1,)), ((), ())),
                                   preferred_element_type=jnp.float32)

        g = jnp.concatenate([mm(x, g0_ref[0, 0]), mm(x, g1_ref[0, 0])], axis=1)
        u = jnp.concatenate([mm(x, u0_ref[0, 0]), mm(x, u1_ref[0, 0])], axis=1)
        inter = g * jax.nn.sigmoid(g) * u
        for k, d_ref in enumerate((d0_ref, d1_ref, d2_ref, d3_ref)):
            o_ref[:, k * HQ:(k + 1) * HQ] = mm(inter, d_ref[0, 0])


def _run_grouped_mlp(be, act, xs, gate_up_w, down_w):
    guw4 = gate_up_w.reshape(E, 4, IH, H)
    dw4 = down_w.reshape(E, 4, HQ, I)

    def wspec(j):
        return pl.BlockSpec((1, 1, IH, H), lambda i, be, act, j=j: (be[i], j, 0, 0))

    def dspec(j):
        return pl.BlockSpec((1, 1, HQ, I), lambda i, be, act, j=j: (be[i], j, 0, 0))

    grid_spec = pltpu.PrefetchScalarGridSpec(
        num_scalar_prefetch=2,
        grid=(NB,),
        in_specs=[
            pl.BlockSpec((BLK, H), lambda i, be, act: (i, 0)),
            wspec(0), wspec(1), wspec(2), wspec(3),
            dspec(0), dspec(1), dspec(2), dspec(3),
        ],
        out_specs=pl.BlockSpec((BLK, H), lambda i, be, act: (i, 0)),
    )
    return pl.pallas_call(
        _mlp_body,
        grid_spec=grid_spec,
        out_shape=jax.ShapeDtypeStruct((PAD, H), jnp.float32),
    )(be, act, xs, guw4, guw4, guw4, guw4, dw4, dw4, dw4, dw4)


SBLK = 256


def _shared_body(x_ref, gw_ref, uw_ref, dw_ref, o_ref):
    x = x_ref[...]
    g = lax.dot_general(x, gw_ref[...], (((1,), (1,)), ((), ())),
                        preferred_element_type=jnp.float32)
    u = lax.dot_general(x, uw_ref[...], (((1,), (1,)), ((), ())),
                        preferred_element_type=jnp.float32)
    inter = g * jax.nn.sigmoid(g) * u
    o_ref[...] = lax.dot_general(inter, dw_ref[...], (((1,), (1,)), ((), ())),
                                 preferred_element_type=jnp.float32)


def _run_shared_mlp(x, gw, uw, dw):
    return pl.pallas_call(
        _shared_body,
        grid=(T // SBLK,),
        in_specs=[
            pl.BlockSpec((SBLK, H), lambda i: (i, 0)),
            pl.BlockSpec((IS, H), lambda i: (0, 0)),
            pl.BlockSpec((IS, H), lambda i: (0, 0)),
            pl.BlockSpec((H, IS), lambda i: (0, 0)),
        ],
        out_specs=pl.BlockSpec((SBLK, H), lambda i: (i, 0)),
        out_shape=jax.ShapeDtypeStruct((T, H), jnp.float32),
    )(x, gw, uw, dw)


NVEC = H // 16
GT = 8
GC = GT * TOPK
NG = (T // NW) // GT


def _sc_combine_body(base_hbm, y_hbm, pos_hbm, w_hbm, out_hbm,
                     posv, wv, buf0, buf1, acc, semb, sem0, sem1):
    wid = lax.axis_index("s") * NC + lax.axis_index("c")
    t0 = wid * (T // NW)
    a0 = wid * A_W
    pltpu.sync_copy(pos_hbm.at[pl.ds(a0, A_W)], posv)
    pltpu.sync_copy(w_hbm.at[pl.ds(a0, A_W)], wv.at[pl.ds(0, A_W)])
    cb = pltpu.async_copy(base_hbm.at[pl.ds(t0, T // NW)], acc, semb)
    bufs = (buf0, buf1)
    sems = (sem0, sem1)
    handles = [None, None]
    handles[0] = pltpu.async_copy(y_hbm.at[posv.at[pl.ds(0, GC)]], buf0, sem0)
    cb.wait()
    for g in range(NG):
        b = g % 2
        nb = (g + 1) % 2
        if g + 1 < NG:
            handles[nb] = pltpu.async_copy(
                y_hbm.at[posv.at[pl.ds((g + 1) * GC, GC)]], bufs[nb], sems[nb])
        handles[b].wait()
        buf = bufs[b]
        wsc = []
        for t in range(GT):
            wt = wv[pl.ds(g * GC + t * TOPK, 16)]
            wsc.append([wt[k] for k in range(TOPK)])

        def cbody(c, _, g=g, buf=buf, wsc=wsc):
            sl = pl.ds(c * 16, 16)
            for t in range(GT):
                av = acc[g * GT + t, sl]
                for k in range(TOPK):
                    av = av + wsc[t][k] * buf[t * TOPK + k, sl]
                acc[g * GT + t, sl] = av
            return 0

        lax.fori_loop(0, NVEC, cbody, 0)
    pltpu.sync_copy(acc, out_hbm.at[pl.ds(t0, T // NW)])


def _run_sc_combine(shared_out, y, pos_flat, w_flat):
    mesh = plsc.VectorSubcoreMesh(core_axis_name="c", subcore_axis_name="s")
    fn = functools.partial(
        pl.kernel,
        out_type=jax.ShapeDtypeStruct((T, H), jnp.float32),
        mesh=mesh,
        scratch_types=[
            pltpu.VMEM((A_W,), jnp.int32),
            pltpu.VMEM((A_W + 16,), jnp.float32),
            pltpu.VMEM((GC, H), jnp.float32),
            pltpu.VMEM((GC, H), jnp.float32),
            pltpu.VMEM((T // NW, H), jnp.float32),
            pltpu.SemaphoreType.DMA,
            pltpu.SemaphoreType.DMA,
            pltpu.SemaphoreType.DMA,
        ],
    )(_sc_combine_body)
    return fn(shared_out, y, pos_flat, w_flat)


def kernel(hidden_states, router_weight, bias, gate_up_w, down_w,
           shared_gate_w, shared_up_w, shared_down_w):
    orig_shape = hidden_states.shape
    x = hidden_states.reshape(T, H)

    topk_w, pos, be8 = _run_router(x, router_weight, bias)
    be = be8[0]
    act = be8[1]
    pos_flat = pos.reshape(A)
    w_flat = topk_w.reshape(A)
    tok_flat = jnp.repeat(jnp.arange(T, dtype=jnp.int32), TOPK)

    shared_out = _run_shared_mlp(x, shared_gate_w, shared_up_w, shared_down_w)
    xs = _run_sc_scatter(x, tok_flat, pos_flat)
    y = _run_grouped_mlp(be, act, xs, gate_up_w, down_w)
    out = _run_sc_combine(shared_out, y, pos_flat, w_flat)

    out = out.reshape(orig_shape)
    return (out, out)


# speedup vs baseline: 1.6766x; 1.0340x over previous
import functools

import jax
import jax.numpy as jnp
from jax import lax
from jax.experimental import pallas as pl
from jax.experimental.pallas import tpu as pltpu
from jax.experimental.pallas import tpu_sc as plsc

H = 768
E = 128
I = 1856
IS = 3712
TOPK = 6
RSF = 2.5
T = 2048

BLK = 128
NB = 224
PAD = NB * BLK

NC = 2
NS = 16
NW = NC * NS
A = T * TOPK
A_W = A // NW
CH = 64
NCH = A_W // CH


def _router_body(x_ref, rw_ref, bias_ref, w_ref, pos_ref, be_ref):
    x = x_ref[...]
    rw = rw_ref[...]
    logits = lax.dot_general(
        x, rw, (((1,), (1,)), ((), ())),
        preferred_element_type=jnp.float32)
    scores = jax.nn.sigmoid(logits)
    sfc = scores + bias_ref[...]

    lane = lax.broadcasted_iota(jnp.int32, (T, E), 1)
    cur = sfc
    onehots = []
    ws = []
    for _ in range(TOPK):
        m = jnp.max(cur, axis=1, keepdims=True)
        idx = jnp.min(jnp.where(cur == m, lane, E), axis=1, keepdims=True)
        oh = lane == idx
        onehots.append(oh)
        ws.append(jnp.sum(jnp.where(oh, scores, 0.0), axis=1, keepdims=True))
        cur = jnp.where(oh, -1e30, cur)

    w = jnp.concatenate(ws, axis=1)
    denom = jnp.sum(w, axis=1, keepdims=True) + 1e-20
    w_ref[...] = w / denom * RSF

    hist = jnp.zeros((T, E), jnp.float32)
    for oh in onehots:
        hist = hist + oh.astype(jnp.float32)

    r128 = lax.broadcasted_iota(jnp.int32, (BLK, BLK), 0)
    c128 = lax.broadcasted_iota(jnp.int32, (BLK, BLK), 1)
    ltri = (r128 > c128).astype(jnp.float32)
    carry = jnp.zeros((1, E), jnp.float32)
    chunks = []
    for c in range(T // BLK):
        h_c = hist[c * BLK:(c + 1) * BLK, :]
        e_c = lax.dot_general(ltri, h_c, (((1,), (0,)), ((), ())),
                              preferred_element_type=jnp.float32) + carry
        chunks.append(e_c)
        carry = carry + jnp.sum(h_c, axis=0, keepdims=True)
    excl = jnp.concatenate(chunks, axis=0)
    counts = carry

    nblk = jnp.ceil(counts / BLK)
    re_ = lax.broadcasted_iota(jnp.int32, (E, E), 0)
    ce_ = lax.broadcasted_iota(jnp.int32, (E, E), 1)
    utri = (re_ < ce_).astype(jnp.float32)
    base_blk = lax.dot_general(nblk, utri, (((1,), (0,)), ((), ())),
                               preferred_element_type=jnp.float32)
    base = base_blk * BLK

    slot = base + excl
    poss = []
    for oh in onehots:
        poss.append(jnp.sum(jnp.where(oh, slot, 0.0), axis=1, keepdims=True))
    pos_ref[...] = jnp.concatenate(poss, axis=1).astype(jnp.int32)

    bstart = lax.broadcasted_iota(jnp.int32, (8, NB), 1).astype(jnp.float32) * BLK
    owner = jnp.zeros((8, NB), jnp.float32)
    for e in range(E):
        owner = owner + (base[0, e] <= bstart).astype(jnp.float32)
    owner = owner - 1.0
    total_rows = jnp.sum(nblk) * BLK
    act = (bstart < total_rows).astype(jnp.float32)
    row = lax.broadcasted_iota(jnp.int32, (8, NB), 0)
    be_ref[...] = jnp.where(row == 1, act, owner).astype(jnp.int32)


def _run_router(x, router_weight, bias):
    return pl.pallas_call(
        _router_body,
        out_shape=(
            jax.ShapeDtypeStruct((T, TOPK), jnp.float32),
            jax.ShapeDtypeStruct((T, TOPK), jnp.int32),
            jax.ShapeDtypeStruct((8, NB), jnp.int32),
        ),
    )(x, router_weight, bias.reshape(1, E))


def _sc_scatter_body(x_hbm, tok_hbm, pos_hbm, xs_hbm,
                     tokv0, tokv1, posv0, posv1, buf0, buf1,
                     sg0, sg1, ss0, ss1):
    wid = lax.axis_index("s") * NC + lax.axis_index("c")
    a0 = wid * A_W
    tokv = (tokv0, tokv1)
    posv = (posv0, posv1)
    bufs = (buf0, buf1)
    gs = (sg0, sg1)
    ss = (ss0, ss1)
    scatters = [None, None]
    pltpu.sync_copy(tok_hbm.at[pl.ds(a0, CH)], tokv0)
    pltpu.sync_copy(pos_hbm.at[pl.ds(a0, CH)], posv0)
    g_prev = pltpu.async_copy(x_hbm.at[tokv0], buf0, sg0)
    for j in range(NCH):
        b = j % 2
        nb = (j + 1) % 2
        if j + 1 < NCH:
            if scatters[nb] is not None:
                scatters[nb].wait()
            pltpu.sync_copy(tok_hbm.at[pl.ds(a0 + (j + 1) * CH, CH)], tokv[nb])
            pltpu.sync_copy(pos_hbm.at[pl.ds(a0 + (j + 1) * CH, CH)], posv[nb])
            g_next = pltpu.async_copy(x_hbm.at[tokv[nb]], bufs[nb], gs[nb])
        g_prev.wait()
        scatters[b] = pltpu.async_copy(bufs[b], xs_hbm.at[posv[b]], ss[b])
        if j + 1 < NCH:
            g_prev = g_next
    scatters[0].wait()
    scatters[1].wait()


def _run_sc_scatter(x, tok_flat, pos_flat):
    mesh = plsc.VectorSubcoreMesh(core_axis_name="c", subcore_axis_name="s")
    fn = functools.partial(
        pl.kernel,
        out_type=jax.ShapeDtypeStruct((PAD, H), jnp.float32),
        mesh=mesh,
        scratch_types=[
            pltpu.VMEM((CH,), jnp.int32),
            pltpu.VMEM((CH,), jnp.int32),
            pltpu.VMEM((CH,), jnp.int32),
            pltpu.VMEM((CH,), jnp.int32),
            pltpu.VMEM((CH, H), jnp.float32),
            pltpu.VMEM((CH, H), jnp.float32),
            pltpu.SemaphoreType.DMA,
            pltpu.SemaphoreType.DMA,
            pltpu.SemaphoreType.DMA,
            pltpu.SemaphoreType.DMA,
        ],
    )(_sc_scatter_body)
    return fn(x, tok_flat, pos_flat)


IH = I // 2
HQ = H // 4


def _mlp_body(be_ref, act_ref, xs_ref, g0_ref, g1_ref, u0_ref, u1_ref,
              d0_ref, d1_ref, d2_ref, d3_ref, o_ref):
    @pl.when(act_ref[pl.program_id(0)] > 0)
    def _():
        x = xs_ref[...]

        def mm(a, b):
            return lax.dot_general(a, b, (((1,), (1,)), ((), ())),
                                   preferred_element_type=jnp.float32)

        g = jnp.concatenate([mm(x, g0_ref[0, 0]), mm(x, g1_ref[0, 0])], axis=1)
        u = jnp.concatenate([mm(x, u0_ref[0, 0]), mm(x, u1_ref[0, 0])], axis=1)
        inter = g * jax.nn.sigmoid(g) * u
        for k, d_ref in enumerate((d0_ref, d1_ref, d2_ref, d3_ref)):
            o_ref[:, k * HQ:(k + 1) * HQ] = mm(inter, d_ref[0, 0])


def _run_grouped_mlp(be, act, xs, gate_up_w, down_w):
    guw4 = gate_up_w.reshape(E, 4, IH, H)
    dw4 = down_w.reshape(E, 4, HQ, I)

    def wspec(j):
        return pl.BlockSpec((1, 1, IH, H), lambda i, be, act, j=j: (be[i], j, 0, 0))

    def dspec(j):
        return pl.BlockSpec((1, 1, HQ, I), lambda i, be, act, j=j: (be[i], j, 0, 0))

    grid_spec = pltpu.PrefetchScalarGridSpec(
        num_scalar_prefetch=2,
        grid=(NB,),
        in_specs=[
            pl.BlockSpec((BLK, H),
                         lambda i, be, act: (jnp.where(act[i] > 0, i, NB - 1), 0)),
            wspec(0), wspec(1), wspec(2), wspec(3),
            dspec(0), dspec(1), dspec(2), dspec(3),
        ],
        out_specs=pl.BlockSpec((BLK, H),
                               lambda i, be, act: (jnp.where(act[i] > 0, i, NB - 1), 0)),
    )
    return pl.pallas_call(
        _mlp_body,
        grid_spec=grid_spec,
        out_shape=jax.ShapeDtypeStruct((PAD, H), jnp.float32),
    )(be, act, xs, guw4, guw4, guw4, guw4, dw4, dw4, dw4, dw4)


SBLK = 256


def _shared_body(x_ref, gw_ref, uw_ref, dw_ref, o_ref):
    x = x_ref[...]
    g = lax.dot_general(x, gw_ref[...], (((1,), (1,)), ((), ())),
                        preferred_element_type=jnp.float32)
    u = lax.dot_general(x, uw_ref[...], (((1,), (1,)), ((), ())),
                        preferred_element_type=jnp.float32)
    inter = g * jax.nn.sigmoid(g) * u
    o_ref[...] = lax.dot_general(inter, dw_ref[...], (((1,), (1,)), ((), ())),
                                 preferred_element_type=jnp.float32)


def _run_shared_mlp(x, gw, uw, dw):
    return pl.pallas_call(
        _shared_body,
        grid=(T // SBLK,),
        in_specs=[
            pl.BlockSpec((SBLK, H), lambda i: (i, 0)),
            pl.BlockSpec((IS, H), lambda i: (0, 0)),
            pl.BlockSpec((IS, H), lambda i: (0, 0)),
            pl.BlockSpec((H, IS), lambda i: (0, 0)),
        ],
        out_specs=pl.BlockSpec((SBLK, H), lambda i: (i, 0)),
        out_shape=jax.ShapeDtypeStruct((T, H), jnp.float32),
    )(x, gw, uw, dw)


NVEC = H // 16
GT = 8
GC = GT * TOPK
NG = (T // NW) // GT


def _sc_combine_body(base_hbm, y_hbm, pos_hbm, w_hbm, out_hbm,
                     posv, wv, buf0, buf1, acc, semb, sem0, sem1):
    wid = lax.axis_index("s") * NC + lax.axis_index("c")
    t0 = wid * (T // NW)
    a0 = wid * A_W
    pltpu.sync_copy(pos_hbm.at[pl.ds(a0, A_W)], posv)
    pltpu.sync_copy(w_hbm.at[pl.ds(a0, A_W)], wv.at[pl.ds(0, A_W)])
    cb = pltpu.async_copy(base_hbm.at[pl.ds(t0, T // NW)], acc, semb)
    bufs = (buf0, buf1)
    sems = (sem0, sem1)
    handles = [None, None]
    handles[0] = pltpu.async_copy(y_hbm.at[posv.at[pl.ds(0, GC)]], buf0, sem0)
    cb.wait()
    for g in range(NG):
        b = g % 2
        nb = (g + 1) % 2
        if g + 1 < NG:
            handles[nb] = pltpu.async_copy(
                y_hbm.at[posv.at[pl.ds((g + 1) * GC, GC)]], bufs[nb], sems[nb])
        handles[b].wait()
        buf = bufs[b]
        wsc = []
        for t in range(GT):
            wt = wv[pl.ds(g * GC + t * TOPK, 16)]
            wsc.append([wt[k] for k in range(TOPK)])

        def cbody(c, _, g=g, buf=buf, wsc=wsc):
            sl = pl.ds(c * 16, 16)
            for t in range(GT):
                av = acc[g * GT + t, sl]
                for k in range(TOPK):
                    av = av + wsc[t][k] * buf[t * TOPK + k, sl]
                acc[g * GT + t, sl] = av
            return 0

        lax.fori_loop(0, NVEC, cbody, 0)
    pltpu.sync_copy(acc, out_hbm.at[pl.ds(t0, T // NW)])


def _run_sc_combine(shared_out, y, pos_flat, w_flat):
    mesh = plsc.VectorSubcoreMesh(core_axis_name="c", subcore_axis_name="s")
    fn = functools.partial(
        pl.kernel,
        out_type=jax.ShapeDtypeStruct((T, H), jnp.float32),
        mesh=mesh,
        scratch_types=[
            pltpu.VMEM((A_W,), jnp.int32),
            pltpu.VMEM((A_W + 16,), jnp.float32),
            pltpu.VMEM((GC, H), jnp.float32),
            pltpu.VMEM((GC, H), jnp.float32),
            pltpu.VMEM((T // NW, H), jnp.float32),
            pltpu.SemaphoreType.DMA,
            pltpu.SemaphoreType.DMA,
            pltpu.SemaphoreType.DMA,
        ],
    )(_sc_combine_body)
    return fn(shared_out, y, pos_flat, w_flat)


def kernel(hidden_states, router_weight, bias, gate_up_w, down_w,
           shared_gate_w, shared_up_w, shared_down_w):
    orig_shape = hidden_states.shape
    x = hidden_states.reshape(T, H)

    topk_w, pos, be8 = _run_router(x, router_weight, bias)
    be = be8[0]
    act = be8[1]
    pos_flat = pos.reshape(A)
    w_flat = topk_w.reshape(A)
    tok_flat = jnp.repeat(jnp.arange(T, dtype=jnp.int32), TOPK)

    shared_out = _run_shared_mlp(x, shared_gate_w, shared_up_w, shared_down_w)
    xs = _run_sc_scatter(x, tok_flat, pos_flat)
    y = _run_grouped_mlp(be, act, xs, gate_up_w, down_w)
    out = _run_sc_combine(shared_out, y, pos_flat, w_flat)

    out = out.reshape(orig_shape)
    return (out, out)
